# baseline (speedup 1.0000x reference)
import functools

import jax
import jax.numpy as jnp
from jax import lax
from jax.experimental import pallas as pl
from jax.experimental.pallas import tpu as pltpu
from jax.experimental.pallas import tpu_sc as plsc

B = 16384
V = 10
D = 784
NC = 2
NS = 16
NW = NC * NS
BPW = B // NW
L = 16
NG = BPW // L
H = 28
SH = 4
NSLAB = H // SH


def _make_sc_gather():
    mesh = plsc.VectorSubcoreMesh(core_axis_name="c", subcore_axis_name="s")

    @functools.partial(
        pl.kernel,
        mesh=mesh,
        compiler_params=pltpu.CompilerParams(
            needs_layout_passes=False, use_tc_tiling_on_sc=False),
        out_type=jax.ShapeDtypeStruct((H, 28, 1, B), jnp.float32),
        scratch_types=[
            pltpu.VMEM((BPW,), jnp.int32),
            pltpu.VMEM((D * L,), jnp.float32),
            pltpu.VMEM((2, SH, 28, 1, BPW), jnp.float32),
            pltpu.SemaphoreType.DMA,
            pltpu.SemaphoreType.DMA,
            pltpu.SemaphoreType.DMA,
            pltpu.SemaphoreType.DMA,
        ],
    )
    def k(idx_hbm, table_hbm, out_hbm, idx_v, table_v, buf_v,
          isem, tsem, osem0, osem1):
        wid = lax.axis_index("s") * NC + lax.axis_index("c")
        base = wid * BPW
        icopy = pltpu.async_copy(idx_hbm.at[pl.ds(base, BPW)], idx_v, isem)
        pltpu.async_copy(table_hbm, table_v, tsem).wait()
        icopy.wait()

        osems = (osem0, osem1)
        outcopies = [None, None]

        for sl in range(NSLAB):
            cur = sl % 2
            if outcopies[cur] is not None:
                outcopies[cur].wait()

            @plsc.parallel_loop(0, NG * SH, unroll=2)
            def body(t):
                g = t // SH
                hh = t % SH
                iv = idx_v[pl.ds(g * L, L)]
                s0 = (sl * SH + hh) * (28 * L)
                for w in range(28):
                    vals = plsc.load_gather(table_v, [iv + (s0 + w * L)])
                    buf_v[cur, hh, w, 0, pl.ds(g * L, L)] = vals
            outcopies[cur] = pltpu.async_copy(
                buf_v.at[cur],
                out_hbm.at[pl.ds(sl * SH, SH), :, :, pl.ds(base, BPW)],
                osems[cur])
        outcopies[0].wait()
        outcopies[1].wait()

    return k


_sc_gather = _make_sc_gather()


def kernel(x, logits):
    table_t = jnp.pad(logits.reshape(V, D).T, ((0, 0), (0, L - V)))
    out_t = _sc_gather(x.astype(jnp.int32), table_t.reshape(D * L))
    return jnp.transpose(out_t, (3, 2, 0, 1))


# speedup vs baseline: 1.1027x; 1.1027x over previous
import functools

import jax
import jax.numpy as jnp
from jax import lax
from jax.experimental import pallas as pl
from jax.experimental.pallas import tpu as pltpu
from jax.experimental.pallas import tpu_sc as plsc

B = 16384
V = 10
D = 784
NC = 2
NS = 16
NW = NC * NS
BPW = B // NW
L = 16
NG = BPW // L
H = 28
SH = 4
NSLAB = H // SH


def _make_sc_gather():
    mesh = plsc.VectorSubcoreMesh(core_axis_name="c", subcore_axis_name="s")

    @functools.partial(
        pl.kernel,
        mesh=mesh,
        compiler_params=pltpu.CompilerParams(
            needs_layout_passes=False, use_tc_tiling_on_sc=False),
        out_type=jax.ShapeDtypeStruct((H, 28, 1, B), jnp.float32),
        scratch_types=[
            pltpu.VMEM((BPW,), jnp.int32),
            pltpu.VMEM((D * L,), jnp.float32),
            pltpu.VMEM((2, SH, 28, 1, BPW), jnp.float32),
            pltpu.SemaphoreType.DMA,
            pltpu.SemaphoreType.DMA,
            pltpu.SemaphoreType.DMA,
            pltpu.SemaphoreType.DMA,
        ],
    )
    def k(idx_hbm, table_hbm, out_hbm, idx_v, table_v, buf_v,
          isem, tsem, osem0, osem1):
        wid = lax.axis_index("s") * NC + lax.axis_index("c")
        base = wid * BPW
        icopy = pltpu.async_copy(idx_hbm.at[pl.ds(base, BPW)], idx_v, isem)
        pltpu.async_copy(table_hbm, table_v, tsem).wait()
        icopy.wait()

        osems = (osem0, osem1)
        outcopies = [None, None]

        for sl in range(NSLAB):
            cur = sl % 2
            if outcopies[cur] is not None:
                outcopies[cur].wait()

            @plsc.parallel_loop(0, NG * SH)
            def body(t):
                g = t // SH
                hh = t % SH
                iv = idx_v[pl.ds(g * L, L)]
                s0 = (sl * SH + hh) * (28 * L)
                for w in range(28):
                    vals = plsc.load_gather(table_v, [iv + (s0 + w * L)])
                    buf_v[cur, hh, w, 0, pl.ds(g * L, L)] = vals
            outcopies[cur] = pltpu.async_copy(
                buf_v.at[cur],
                out_hbm.at[pl.ds(sl * SH, SH), :, :, pl.ds(base, BPW)],
                osems[cur])
        outcopies[0].wait()
        outcopies[1].wait()

    return k


_sc_gather = _make_sc_gather()


def kernel(x, logits):
    table_t = jnp.pad(logits.reshape(V, D).T, ((0, 0), (0, L - V)))
    out_t = _sc_gather(x.astype(jnp.int32), table_t.reshape(D * L))
    return jnp.transpose(out_t, (3, 2, 0, 1))
